# baseline (speedup 1.0000x reference)
import functools

import jax
import jax.numpy as jnp
from jax.experimental import pallas as pl
from jax.experimental.pallas import tpu as pltpu

_BI1 = 200
_BI2 = 1000


def _pass1_kernel(l_ref, xt_ref, lb_ref, x0_ref, x1_ref):
    i = pl.program_id(0)

    @pl.when(i == 0)
    def _():
        x0_ref[...] = jnp.transpose(xt_ref[...]).astype(jnp.bfloat16)

    lb = l_ref[...].astype(jnp.bfloat16)
    lb_ref[...] = lb
    acc = jnp.dot(lb, x0_ref[...], preferred_element_type=jnp.float32)
    x1_ref[...] = acc.astype(jnp.bfloat16)


def _pass23_kernel(lb_ref, x0_ref, x1_ref, w_ref, b_ref, out_ref, x2_ref,
                   *, bi):
    k = pl.program_id(0)
    i = pl.program_id(1)
    isl = pl.ds(i * bi, bi)

    @pl.when(k == 0)
    def _():
        acc = jnp.dot(lb_ref[...], x1_ref[...],
                      preferred_element_type=jnp.float32)
        x2 = 2.0 * acc - x0_ref[...].astype(jnp.float32)
        x2_ref[isl, :] = x2.astype(jnp.bfloat16)

    @pl.when(k == 1)
    def _():
        acc = jnp.dot(lb_ref[...], x2_ref[...],
                      preferred_element_type=jnp.float32)
        x3 = 2.0 * acc - x1_ref[isl, :].astype(jnp.float32)
        w = w_ref[...]
        o = jnp.dot(x3.astype(jnp.bfloat16), w[3],
                    preferred_element_type=jnp.float32)
        o += jnp.dot(x0_ref[...], w[0], preferred_element_type=jnp.float32)
        o += jnp.dot(x1_ref[isl, :], w[1], preferred_element_type=jnp.float32)
        o += jnp.dot(x2_ref[isl, :], w[2], preferred_element_type=jnp.float32)
        out_ref[...] = o + b_ref[...]


def kernel(x, laplacian, weights, biases):
    B, Cin, V = x.shape
    K, _, Cout = weights.shape
    BC = B * Cin
    BCO = B * Cout

    xt = x.reshape(BC, V)

    eye_b = jnp.eye(B, dtype=weights.dtype)
    wbig = jnp.einsum("bd,kio->kbido", eye_b, weights).reshape(K, BC, BCO)
    wbig = wbig.astype(jnp.bfloat16)
    bbig = jnp.tile(biases, B)[None, :]

    ni1 = V // _BI1
    lb, x0b, x1 = pl.pallas_call(
        _pass1_kernel,
        grid=(ni1,),
        in_specs=[
            pl.BlockSpec((_BI1, V), lambda i: (i, 0)),
            pl.BlockSpec((BC, V), lambda i: (0, 0)),
        ],
        out_specs=[
            pl.BlockSpec((_BI1, V), lambda i: (i, 0)),
            pl.BlockSpec((V, BC), lambda i: (0, 0)),
            pl.BlockSpec((_BI1, BC), lambda i: (i, 0)),
        ],
        out_shape=[
            jax.ShapeDtypeStruct((V, V), jnp.bfloat16),
            jax.ShapeDtypeStruct((V, BC), jnp.bfloat16),
            jax.ShapeDtypeStruct((V, BC), jnp.bfloat16),
        ],
        compiler_params=pltpu.CompilerParams(
            dimension_semantics=("arbitrary",),
            vmem_limit_bytes=64 * 1024 * 1024,
        ),
    )(laplacian, xt)

    ni2 = V // _BI2
    body = functools.partial(_pass23_kernel, bi=_BI2)
    out2 = pl.pallas_call(
        body,
        grid=(2, ni2),
        in_specs=[
            pl.BlockSpec((_BI2, V), lambda k, i: (i, 0)),
            pl.BlockSpec((_BI2, BC), lambda k, i: (i, 0)),
            pl.BlockSpec((V, BC), lambda k, i: (0, 0)),
            pl.BlockSpec((K, BC, BCO), lambda k, i: (0, 0, 0)),
            pl.BlockSpec((1, BCO), lambda k, i: (0, 0)),
        ],
        out_specs=pl.BlockSpec((_BI2, BCO), lambda k, i: (i, 0)),
        out_shape=jax.ShapeDtypeStruct((V, BCO), jnp.float32),
        scratch_shapes=[
            pltpu.VMEM((V, BC), jnp.bfloat16),
        ],
        compiler_params=pltpu.CompilerParams(
            dimension_semantics=("arbitrary", "arbitrary"),
            vmem_limit_bytes=62 * 1024 * 1024,
        ),
    )(lb, x0b, x1, wbig, bbig)

    return jnp.transpose(out2.reshape(V, B, Cout), (1, 2, 0))


# speedup vs baseline: 1.0912x; 1.0912x over previous
import functools

import jax
import jax.numpy as jnp
from jax.experimental import pallas as pl
from jax.experimental.pallas import tpu as pltpu

_BI1 = 400
_BI2 = 1000


def _pass1_kernel(l_ref, x0_ref, lb_ref, x1_ref):
    lb = l_ref[...].astype(jnp.bfloat16)
    lb_ref[...] = lb
    acc = jnp.dot(lb, x0_ref[...], preferred_element_type=jnp.float32)
    x1_ref[...] = acc.astype(jnp.bfloat16)


def _pass23_kernel(lb_ref, x0_ref, x1_ref, w_ref, b_ref, out_ref, x2_ref,
                   *, bi):
    k = pl.program_id(0)
    i = pl.program_id(1)
    isl = pl.ds(i * bi, bi)

    @pl.when(k == 0)
    def _():
        acc = jnp.dot(lb_ref[...], x1_ref[...],
                      preferred_element_type=jnp.float32)
        x2 = 2.0 * acc - x0_ref[isl, :].astype(jnp.float32)
        x2_ref[isl, :] = x2.astype(jnp.bfloat16)

    @pl.when(k == 1)
    def _():
        acc = jnp.dot(lb_ref[...], x2_ref[...],
                      preferred_element_type=jnp.float32)
        x3 = 2.0 * acc - x1_ref[isl, :].astype(jnp.float32)
        w = w_ref[...]
        o = jnp.dot(x3.astype(jnp.bfloat16), w[3],
                    preferred_element_type=jnp.float32)
        o += jnp.dot(x0_ref[isl, :], w[0], preferred_element_type=jnp.float32)
        o += jnp.dot(x1_ref[isl, :], w[1], preferred_element_type=jnp.float32)
        o += jnp.dot(x2_ref[isl, :], w[2], preferred_element_type=jnp.float32)
        out_ref[...] = o + b_ref[...]


def kernel(x, laplacian, weights, biases):
    B, Cin, V = x.shape
    K, _, Cout = weights.shape
    BC = B * Cin
    BCO = B * Cout

    x0 = jnp.transpose(x, (2, 0, 1)).reshape(V, BC).astype(jnp.bfloat16)

    eye_b = jnp.eye(B, dtype=weights.dtype)
    wbig = jnp.einsum("bd,kio->kbido", eye_b, weights).reshape(K, BC, BCO)
    wbig = wbig.astype(jnp.bfloat16)
    bbig = jnp.tile(biases, B)[None, :]

    ni1 = V // _BI1
    lb, x1 = pl.pallas_call(
        _pass1_kernel,
        grid=(ni1,),
        in_specs=[
            pl.BlockSpec((_BI1, V), lambda i: (i, 0)),
            pl.BlockSpec((V, BC), lambda i: (0, 0)),
        ],
        out_specs=[
            pl.BlockSpec((_BI1, V), lambda i: (i, 0)),
            pl.BlockSpec((_BI1, BC), lambda i: (i, 0)),
        ],
        out_shape=[
            jax.ShapeDtypeStruct((V, V), jnp.bfloat16),
            jax.ShapeDtypeStruct((V, BC), jnp.bfloat16),
        ],
        compiler_params=pltpu.CompilerParams(
            dimension_semantics=("arbitrary",),
            vmem_limit_bytes=62 * 1024 * 1024,
        ),
    )(laplacian, x0)

    ni2 = V // _BI2
    body = functools.partial(_pass23_kernel, bi=_BI2)
    out2 = pl.pallas_call(
        body,
        grid=(2, ni2),
        in_specs=[
            pl.BlockSpec((_BI2, V), lambda k, i: (i, 0)),
            pl.BlockSpec((V, BC), lambda k, i: (0, 0)),
            pl.BlockSpec((V, BC), lambda k, i: (0, 0)),
            pl.BlockSpec((K, BC, BCO), lambda k, i: (0, 0, 0)),
            pl.BlockSpec((1, BCO), lambda k, i: (0, 0)),
        ],
        out_specs=pl.BlockSpec((_BI2, BCO), lambda k, i: (i, 0)),
        out_shape=jax.ShapeDtypeStruct((V, BCO), jnp.float32),
        scratch_shapes=[
            pltpu.VMEM((V, BC), jnp.bfloat16),
        ],
        compiler_params=pltpu.CompilerParams(
            dimension_semantics=("arbitrary", "arbitrary"),
            vmem_limit_bytes=62 * 1024 * 1024,
        ),
    )(lb, x0, x1, wbig, bbig)

    return jnp.transpose(out2.reshape(V, B, Cout), (1, 2, 0))
